# baseline (speedup 1.0000x reference)
import functools

import jax
import jax.numpy as jnp
import numpy as np
from jax import lax
from jax.experimental import pallas as pl
from jax.experimental.pallas import tpu as pltpu
from jax.experimental.pallas import tpu_sc as plsc

N, S, D, C, TOPK = 64, 2048, 128, 1024, 64
NB = 8
NWORKERS = 32
ROWS_PER_W = N // NWORKERS
INT_MIN = np.int32(-2147483648)


def _prefix_exclusive(x):
    NCH = S // 128
    x3 = x.reshape(NB, NCH, 128)
    lane = lax.broadcasted_iota(jnp.int32, (NB, NCH, 128), 2)
    c = x3
    for sft in (1, 2, 4, 8, 16, 32, 64):
        c = c + jnp.where(lane >= sft, pltpu.roll(c, sft, 2), jnp.int32(0))
    tot = c[:, :, 127]
    ch = lax.broadcasted_iota(jnp.int32, (NB, NCH), 1)
    t2 = tot
    for sft in (1, 2, 4, 8):
        t2 = t2 + jnp.where(ch >= sft, pltpu.roll(t2, sft, 1), jnp.int32(0))
    excl = t2 - tot
    return (c - x3 + excl[:, :, None]).reshape(NB, S)


def _logits_select_body(q_ref, K_ref, w_ref, dst_ref):
    rows = [
        lax.dot_general(q_ref[i:i + 1, :], K_ref[i],
                        (((1,), (1,)), ((), ())),
                        preferred_element_type=jnp.float32)
        for i in range(NB)
    ]
    L = jnp.concatenate(rows, axis=0) * np.float32(1.0 / np.sqrt(D))

    b = lax.bitcast_convert_type(L, jnp.int32)
    key = b ^ jnp.where(b < 0, jnp.int32(0x7FFFFFFF), jnp.int32(0))

    def count_ge(t):
        return jnp.sum(jnp.where(key >= t, jnp.int32(1), jnp.int32(0)),
                       axis=-1, keepdims=True)

    T = jnp.where(count_ge(jnp.zeros((NB, 1), jnp.int32)) >= TOPK,
                  jnp.int32(0), INT_MIN)
    for bit in range(30, -1, -1):
        Tc = T | jnp.int32(1 << bit)
        T = jnp.where(count_ge(Tc) >= TOPK, Tc, T)

    gt = key > T
    eq = key == T
    cnt_gt = jnp.sum(jnp.where(gt, jnp.int32(1), jnp.int32(0)),
                     axis=-1, keepdims=True)
    r = TOPK - cnt_gt
    a = jnp.where(gt, jnp.int32(65536), jnp.int32(0)) \
        + jnp.where(eq, jnp.int32(1), jnp.int32(0))
    P = _prefix_exclusive(a)
    eqb = P & jnp.int32(0xFFFF)
    gtb = lax.shift_right_logical(P, 16)
    sel = gt | (eq & (eqb < r))

    rowmax = jnp.max(L, axis=-1, keepdims=True)
    ex = jnp.where(sel, jnp.exp(L - rowmax), 0.0)
    denom = jnp.sum(ex, axis=-1, keepdims=True)
    w_ref[...] = ex / denom
    srank = gtb + jnp.minimum(eqb, r)
    dst_ref[...] = jnp.where(sel, srank, jnp.int32(0))


def _sc_body_gen(rows_per_w, row_off):
    def body(W_hbm, R_hbm, V_hbm, G_hbm, wk_hbm,
             wrow_v, rrow_v, idx_v, wkv_v, rows_v,
             sem_in, sem_g, sem_out):
        return _sc_work(row_off, rows_per_w,
                        W_hbm, R_hbm, V_hbm, G_hbm, wk_hbm,
                        wrow_v, rrow_v, idx_v, wkv_v, rows_v,
                        sem_in, sem_g, sem_out)
    return body


def _sc_work(row_off, ROWS_PER_W,
             W_hbm, R_hbm, V_hbm, G_hbm, wk_hbm,
             wrow_v, rrow_v, idx_v, wkv_v, rows_v,
             sem_in, sem_g, sem_out):
    wid = lax.axis_index("s") * 2 + lax.axis_index("c")
    lanes = lax.iota(jnp.int32, 16)
    UNROLL = 4
    n0 = wid * ROWS_PER_W
    cps = []
    for rr in range(ROWS_PER_W):
        cps.append(pltpu.async_copy(W_hbm.at[n0 + rr], wrow_v.at[rr], sem_in))
        cps.append(pltpu.async_copy(R_hbm.at[n0 + rr], rrow_v.at[rr], sem_in))
    for rr in range(ROWS_PER_W):
        for j in range(TOPK // 16):
            idx_v[rr, pl.ds(16 * j, 16)] = jnp.zeros((16,), jnp.int32)
            wkv_v[rr, pl.ds(16 * j, 16)] = jnp.zeros((16,), jnp.float32)
    for cp in cps:
        cp.wait()

    gathers = []
    for rr in range(ROWS_PER_W):
        n = n0 + rr

        def chunk(cc, carry, rr=rr, n=n):
            for u in range(UNROLL):
                c = cc * UNROLL + u
                w16 = wrow_v[rr, pl.ds(c * 16, 16)]
                d16 = rrow_v[rr, pl.ds(c * 16, 16)]
                m = w16 > 0.0
                spos = (n + row_off) * S + c * 16 + lanes
                plsc.store_scatter(idx_v.at[rr], [d16], spos, mask=m)
                plsc.store_scatter(wkv_v.at[rr], [d16], w16, mask=m)
            return carry

        lax.fori_loop(0, S // 16 // UNROLL, chunk, jnp.int32(0))
        gathers.append(
            pltpu.async_copy(V_hbm.at[idx_v.at[rr]], rows_v.at[rr], sem_g))

    outs = []
    for rr in range(ROWS_PER_W):
        n = n0 + rr
        gathers[rr].wait()
        outs.append(pltpu.async_copy(
            rows_v.at[rr], G_hbm.at[pl.ds(n * TOPK, TOPK)], sem_out))
        outs.append(pltpu.async_copy(wkv_v.at[rr], wk_hbm.at[n], sem_out))
    for cp in outs:
        cp.wait()


def _make_sc_compact_gather(nrows, row_off):
    rpw = nrows // NWORKERS
    return functools.partial(
        pl.kernel,
        mesh=plsc.VectorSubcoreMesh(core_axis_name="c", subcore_axis_name="s"),
        compiler_params=pltpu.CompilerParams(needs_layout_passes=False),
        out_type=[
            jax.ShapeDtypeStruct((nrows * TOPK, D), jnp.float32),
            jax.ShapeDtypeStruct((nrows, TOPK), jnp.float32),
        ],
        scratch_types=[
            pltpu.VMEM((rpw, S), jnp.float32),
            pltpu.VMEM((rpw, S), jnp.int32),
            pltpu.VMEM((rpw, TOPK), jnp.int32),
            pltpu.VMEM((rpw, TOPK), jnp.float32),
            pltpu.VMEM((rpw, TOPK, D), jnp.float32),
            pltpu.SemaphoreType.DMA,
            pltpu.SemaphoreType.DMA,
            pltpu.SemaphoreType.DMA,
        ],
    )(_sc_body_gen(rpw, row_off))


def _readout_body(wk0_ref, G0_ref, wk1_ref, G1_ref,
                  Wc_ref, bc_ref, Wr_ref, br_ref, cls_ref, rec_ref):
    H = N // 2
    s0 = jnp.sum(G0_ref[...].reshape(H, TOPK, D)
                 * wk0_ref[...][:, :, None], axis=1)
    s1 = jnp.sum(G1_ref[...].reshape(H, TOPK, D)
                 * wk1_ref[...][:, :, None], axis=1)
    s = jnp.concatenate([s0, s1], axis=0)
    cls_ref[...] = lax.dot_general(
        s, Wc_ref[...], (((1,), (1,)), ((), ())),
        preferred_element_type=jnp.float32) + bc_ref[...]
    rec_ref[...] = lax.dot_general(
        s, Wr_ref[...], (((1,), (1,)), ((), ())),
        preferred_element_type=jnp.float32) + br_ref[...]


def _tc1_half(q, K, off_blocks):
    H = N // 2
    return pl.pallas_call(
        _logits_select_body,
        grid=(H // NB,),
        in_specs=[
            pl.BlockSpec((NB, D), lambda i, o=off_blocks: (i + o, 0)),
            pl.BlockSpec((NB, S, D), lambda i, o=off_blocks: (i + o, 0, 0)),
        ],
        out_specs=[
            pl.BlockSpec((NB, S), lambda i: (i, 0)),
            pl.BlockSpec((NB, S), lambda i: (i, 0)),
        ],
        out_shape=[
            jax.ShapeDtypeStruct((H, S), jnp.float32),
            jax.ShapeDtypeStruct((H, S), jnp.int32),
        ],
    )(q, K)


@jax.jit
def kernel(q, K, V, z, y, W_c, b_c, W_r, b_r):
    del z, y
    H = N // 2
    w0, d0 = _tc1_half(q, K, 0)
    w1, d1 = _tc1_half(q, K, H // NB)

    Vf = V.reshape(N * S, D)
    G0, wk0 = _make_sc_compact_gather(H, 0)(w0, d0, Vf)
    G1, wk1 = _make_sc_compact_gather(H, H)(w1, d1, Vf)

    cls_out, rec_out = pl.pallas_call(
        _readout_body,
        in_specs=[
            pl.BlockSpec((H, TOPK), lambda: (0, 0)),
            pl.BlockSpec((H * TOPK, D), lambda: (0, 0)),
            pl.BlockSpec((H, TOPK), lambda: (0, 0)),
            pl.BlockSpec((H * TOPK, D), lambda: (0, 0)),
            pl.BlockSpec((C, D), lambda: (0, 0)),
            pl.BlockSpec((1, C), lambda: (0, 0)),
            pl.BlockSpec((D, D), lambda: (0, 0)),
            pl.BlockSpec((1, D), lambda: (0, 0)),
        ],
        out_specs=[
            pl.BlockSpec((N, C), lambda: (0, 0)),
            pl.BlockSpec((N, D), lambda: (0, 0)),
        ],
        out_shape=[
            jax.ShapeDtypeStruct((N, C), jnp.float32),
            jax.ShapeDtypeStruct((N, D), jnp.float32),
        ],
    )(wk0, G0, wk1, G1, W_c, b_c.reshape(1, C), W_r, b_r.reshape(1, D))

    weights = jnp.concatenate([w0, w1], axis=0)
    return (cls_out, rec_out, weights)


# speedup vs baseline: 1.0584x; 1.0584x over previous
import functools

import jax
import jax.numpy as jnp
import numpy as np
from jax import lax
from jax.experimental import pallas as pl
from jax.experimental.pallas import tpu as pltpu
from jax.experimental.pallas import tpu_sc as plsc

N, S, D, C, TOPK = 64, 2048, 128, 1024, 64
NB = 8
NWORKERS = 32
ROWS_PER_W = N // NWORKERS
INT_MIN = np.int32(-2147483648)


def _prefix_exclusive(x):
    NCH = S // 128
    x3 = x.reshape(NB, NCH, 128)
    lane = lax.broadcasted_iota(jnp.int32, (NB, NCH, 128), 2)
    c = x3
    for sft in (1, 2, 4, 8, 16, 32, 64):
        c = c + jnp.where(lane >= sft, pltpu.roll(c, sft, 2), jnp.int32(0))
    tot = c[:, :, 127]
    ch = lax.broadcasted_iota(jnp.int32, (NB, NCH), 1)
    t2 = tot
    for sft in (1, 2, 4, 8):
        t2 = t2 + jnp.where(ch >= sft, pltpu.roll(t2, sft, 1), jnp.int32(0))
    excl = t2 - tot
    return (c - x3 + excl[:, :, None]).reshape(NB, S)


def _logits_select_body(q_ref, K_ref, w_ref, dst_ref):
    rows = [
        lax.dot_general(q_ref[i:i + 1, :], K_ref[i],
                        (((1,), (1,)), ((), ())),
                        preferred_element_type=jnp.float32)
        for i in range(NB)
    ]
    L = jnp.concatenate(rows, axis=0) * np.float32(1.0 / np.sqrt(D))

    b = lax.bitcast_convert_type(L, jnp.int32)
    key = b ^ jnp.where(b < 0, jnp.int32(0x7FFFFFFF), jnp.int32(0))

    def count_ge(t):
        return jnp.sum(jnp.where(key >= t, jnp.int32(1), jnp.int32(0)),
                       axis=-1, keepdims=True)

    T = jnp.where(count_ge(jnp.zeros((NB, 1), jnp.int32)) >= TOPK,
                  jnp.int32(0), INT_MIN)
    for bit in range(30, -1, -1):
        Tc = T | jnp.int32(1 << bit)
        T = jnp.where(count_ge(Tc) >= TOPK, Tc, T)

    gt = key > T
    eq = key == T
    cnt_gt = jnp.sum(jnp.where(gt, jnp.int32(1), jnp.int32(0)),
                     axis=-1, keepdims=True)
    r = TOPK - cnt_gt
    a = jnp.where(gt, jnp.int32(65536), jnp.int32(0)) \
        + jnp.where(eq, jnp.int32(1), jnp.int32(0))
    P = _prefix_exclusive(a)
    eqb = P & jnp.int32(0xFFFF)
    gtb = lax.shift_right_logical(P, 16)
    sel = gt | (eq & (eqb < r))

    rowmax = jnp.max(L, axis=-1, keepdims=True)
    ex = jnp.where(sel, jnp.exp(L - rowmax), 0.0)
    denom = jnp.sum(ex, axis=-1, keepdims=True)
    w_ref[...] = ex / denom
    srank = gtb + jnp.minimum(eqb, r)
    dst_ref[...] = jnp.where(sel, srank, jnp.int32(0))


def _sc_body(W_hbm, R_hbm, V_hbm, G_hbm, wk_hbm,
             wrow_v, rrow_v, idx_v, wkv_v, rows_v,
             sem_in, sem_g, sem_out):
    wid = lax.axis_index("s") * 2 + lax.axis_index("c")
    lanes = lax.iota(jnp.int32, 16)
    UNROLL = 4
    n0 = wid * ROWS_PER_W
    cps = []
    for rr in range(ROWS_PER_W):
        cps.append(pltpu.async_copy(W_hbm.at[n0 + rr], wrow_v.at[rr], sem_in))
        cps.append(pltpu.async_copy(R_hbm.at[n0 + rr], rrow_v.at[rr], sem_in))
    for rr in range(ROWS_PER_W):
        for j in range(TOPK // 16):
            idx_v[rr, pl.ds(16 * j, 16)] = jnp.zeros((16,), jnp.int32)
            wkv_v[rr, pl.ds(16 * j, 16)] = jnp.zeros((16,), jnp.float32)
    for cp in cps:
        cp.wait()

    gathers = []
    for rr in range(ROWS_PER_W):
        n = n0 + rr

        def chunk(cc, carry, rr=rr, n=n):
            for u in range(UNROLL):
                c = cc * UNROLL + u
                w16 = wrow_v[rr, pl.ds(c * 16, 16)]
                d16 = rrow_v[rr, pl.ds(c * 16, 16)]
                m = w16 > 0.0
                spos = n * S + c * 16 + lanes
                plsc.store_scatter(idx_v.at[rr], [d16], spos, mask=m)
                plsc.store_scatter(wkv_v.at[rr], [d16], w16, mask=m)
            return carry

        lax.fori_loop(0, S // 16 // UNROLL, chunk, jnp.int32(0))
        gathers.append(
            pltpu.async_copy(V_hbm.at[idx_v.at[rr]], rows_v.at[rr], sem_g))

    outs = []
    for rr in range(ROWS_PER_W):
        n = n0 + rr
        gathers[rr].wait()
        outs.append(pltpu.async_copy(
            rows_v.at[rr], G_hbm.at[pl.ds(n * TOPK, TOPK)], sem_out))
        outs.append(pltpu.async_copy(wkv_v.at[rr], wk_hbm.at[n], sem_out))
    for cp in outs:
        cp.wait()


def _make_sc_compact_gather():
    return functools.partial(
        pl.kernel,
        mesh=plsc.VectorSubcoreMesh(core_axis_name="c", subcore_axis_name="s"),
        compiler_params=pltpu.CompilerParams(needs_layout_passes=False),
        out_type=[
            jax.ShapeDtypeStruct((N * TOPK, D), jnp.float32),
            jax.ShapeDtypeStruct((N, TOPK), jnp.float32),
        ],
        scratch_types=[
            pltpu.VMEM((ROWS_PER_W, S), jnp.float32),
            pltpu.VMEM((ROWS_PER_W, S), jnp.int32),
            pltpu.VMEM((ROWS_PER_W, TOPK), jnp.int32),
            pltpu.VMEM((ROWS_PER_W, TOPK), jnp.float32),
            pltpu.VMEM((ROWS_PER_W, TOPK, D), jnp.float32),
            pltpu.SemaphoreType.DMA,
            pltpu.SemaphoreType.DMA,
            pltpu.SemaphoreType.DMA,
        ],
    )(_sc_body)


def _readout_body(wk_ref, G_ref, Wc_ref, bc_ref, Wr_ref, br_ref,
                  cls_ref, rec_ref):
    G = G_ref[...].reshape(N, TOPK, D)
    wk = wk_ref[...]
    s = jnp.sum(G * wk[:, :, None], axis=1)
    cls_ref[...] = lax.dot_general(
        s, Wc_ref[...], (((1,), (1,)), ((), ())),
        preferred_element_type=jnp.float32) + bc_ref[...]
    rec_ref[...] = lax.dot_general(
        s, Wr_ref[...], (((1,), (1,)), ((), ())),
        preferred_element_type=jnp.float32) + br_ref[...]


@jax.jit
def kernel(q, K, V, z, y, W_c, b_c, W_r, b_r):
    del z, y
    weights, dstr = pl.pallas_call(
        _logits_select_body,
        grid=(N // NB,),
        in_specs=[
            pl.BlockSpec((NB, D), lambda i: (i, 0)),
            pl.BlockSpec((NB, S, D), lambda i: (i, 0, 0)),
        ],
        out_specs=[
            pl.BlockSpec((NB, S), lambda i: (i, 0)),
            pl.BlockSpec((NB, S), lambda i: (i, 0)),
        ],
        out_shape=[
            jax.ShapeDtypeStruct((N, S), jnp.float32),
            jax.ShapeDtypeStruct((N, S), jnp.int32),
        ],
    )(q, K)

    G, wk = _make_sc_compact_gather()(weights, dstr, V.reshape(N * S, D))

    cls_out, rec_out = pl.pallas_call(
        _readout_body,
        in_specs=[
            pl.BlockSpec((N, TOPK), lambda: (0, 0)),
            pl.BlockSpec((N * TOPK, D), lambda: (0, 0)),
            pl.BlockSpec((C, D), lambda: (0, 0)),
            pl.BlockSpec((1, C), lambda: (0, 0)),
            pl.BlockSpec((D, D), lambda: (0, 0)),
            pl.BlockSpec((1, D), lambda: (0, 0)),
        ],
        out_specs=[
            pl.BlockSpec((N, C), lambda: (0, 0)),
            pl.BlockSpec((N, D), lambda: (0, 0)),
        ],
        out_shape=[
            jax.ShapeDtypeStruct((N, C), jnp.float32),
            jax.ShapeDtypeStruct((N, D), jnp.float32),
        ],
    )(wk, G, W_c, b_c.reshape(1, C), W_r, b_r.reshape(1, D))

    return (cls_out, rec_out, weights)
